# baseline (speedup 1.0000x reference)
import functools

import jax
import jax.numpy as jnp
from jax import lax
from jax.experimental import pallas as pl
from jax.experimental.pallas import tpu as pltpu
from jax.experimental.pallas import tpu_sc as plsc

N = 10000
FD = 128
EDGES = 320000
NCLS = 40

NC = 2
NS = 16
NW = NC * NS
NP = 10240
RT = NP // NS
DUMP = 10200

B = 128
EW = EDGES // NW
NCH = 80
NBUF = 2
NSI = 4
NOUT = NCH // NSI

BD = 100
NCHD = EW // BD
ZR = 128

_MESH = plsc.VectorSubcoreMesh(core_axis_name="c", subcore_axis_name="s")


@functools.partial(
    pl.kernel,
    out_type=jax.ShapeDtypeStruct((2, NP, FD), jnp.float32),
    mesh=_MESH,
    scratch_types=[
        pltpu.VMEM((NCHD, BD), jnp.int32),
        pltpu.VMEM((BD, FD), jnp.float32),
        pltpu.VMEM((ZR, FD), jnp.float32),
        pltpu.VMEM_SHARED((NP, FD), jnp.float32),
    ],
)
def _sc_degree(dst_hbm, out_hbm, didx_v, ones_v, zb_v, deg_sh):
    c = lax.axis_index("c")
    s = lax.axis_index("s")
    wid = s * NC + c
    pltpu.sync_copy(dst_hbm.at[wid], didx_v)

    def _fill_ones(i, _):
        for j in range(FD // 16):
            ones_v[i, pl.ds(16 * j, 16)] = jnp.ones((16,), jnp.float32)
        return 0

    def _fill_zero(i, _):
        for j in range(FD // 16):
            zb_v[i, pl.ds(16 * j, 16)] = jnp.zeros((16,), jnp.float32)
        return 0

    lax.fori_loop(0, BD, _fill_ones, 0)
    lax.fori_loop(0, ZR, _fill_zero, 0)

    r0 = s * RT
    for k in range(RT // ZR):
        pltpu.sync_copy(zb_v, deg_sh.at[pl.ds(r0 + k * ZR, ZR)])
    plsc.subcore_barrier()

    def _chunk(i, _):
        pltpu.sync_copy(ones_v, deg_sh.at[didx_v.at[i]], add=True)
        return 0

    lax.fori_loop(0, NCHD, _chunk, 0)
    plsc.subcore_barrier()
    pltpu.sync_copy(deg_sh.at[pl.ds(r0, RT)], out_hbm.at[c, pl.ds(r0, RT)])


@functools.partial(
    pl.kernel,
    out_type=jax.ShapeDtypeStruct((2, NP, FD), jnp.float32),
    mesh=_MESH,
    scratch_types=[
        pltpu.VMEM((NCH, B), jnp.int32),
        pltpu.VMEM((NCH, B), jnp.int32),
        pltpu.VMEM((B, FD), jnp.float32),
        pltpu.VMEM_SHARED((NP, FD), jnp.float32),
        pltpu.SemaphoreType.DMA,
    ],
)
def _sc_propagate(u_hbm, src_hbm, dst_hbm, out_hbm,
                  sidx_v, didx_v, rows_v, acc_sh, semg):
    c = lax.axis_index("c")
    s = lax.axis_index("s")
    wid = s * NC + c

    pltpu.sync_copy(src_hbm.at[wid], sidx_v)
    pltpu.sync_copy(dst_hbm.at[wid], didx_v)

    def _zfill(i, _):
        for j in range(FD // 16):
            rows_v[i, pl.ds(16 * j, 16)] = jnp.zeros((16,), jnp.float32)
        return 0

    lax.fori_loop(0, 80, _zfill, 0)

    r0 = s * RT
    for k in range(RT // 80):
        pltpu.sync_copy(rows_v.at[pl.ds(0, 80)],
                        acc_sh.at[pl.ds(r0 + k * 80, 80)])
    plsc.subcore_barrier()

    def _chunk(i, _):
        pltpu.async_copy(u_hbm.at[sidx_v.at[i]], rows_v, semg).wait()
        pltpu.sync_copy(rows_v, acc_sh.at[didx_v.at[i]], add=True)
        return 0

    lax.fori_loop(0, NCH, _chunk, 0)
    plsc.subcore_barrier()
    pltpu.sync_copy(acc_sh.at[pl.ds(r0, RT)], out_hbm.at[c, pl.ds(r0, RT)])


R = 2000
GRID = N // R


def _rows(width):
    return pl.BlockSpec((R, width), lambda i: (i, 0))


def _part(width, which):
    return pl.BlockSpec((1, R, width), lambda i, w=which: (w, i, 0))


def _full(shape):
    return pl.BlockSpec(shape, lambda i: (0,) * len(shape))


def _prep_body(x_ref, dega_ref, degb_ref, w_ref, y_ref, u_ref, d_ref):
    deg = dega_ref[0, :, 0:1] + degb_ref[0, :, 0:1]
    dinv = jnp.where(deg > 0.0, lax.rsqrt(jnp.maximum(deg, 1e-12)), 0.0)
    dinvb = jnp.broadcast_to(dinv, (R, FD))
    x = x_ref[...]
    y_ref[...] = jnp.dot(x, w_ref[...], preferred_element_type=jnp.float32)
    u_ref[...] = dinvb * x
    d_ref[...] = dinvb


_tc_prep = pl.pallas_call(
    _prep_body,
    grid=(GRID,),
    in_specs=[_rows(FD), _part(FD, 0), _part(FD, 1), _full((FD, FD))],
    out_specs=[_rows(FD), _rows(FD), _rows(FD)],
    out_shape=[jax.ShapeDtypeStruct((N, FD), jnp.float32)] * 3,
)


def _step_body(pa_ref, pb_ref, d_ref, w_ref, yin_ref, y_ref, u_ref):
    d = d_ref[...]
    h = d * (pa_ref[0] + pb_ref[0])
    y_ref[...] = yin_ref[...] + jnp.dot(
        h, w_ref[...], preferred_element_type=jnp.float32)
    u_ref[...] = d * h


_tc_step = pl.pallas_call(
    _step_body,
    grid=(GRID,),
    in_specs=[_part(FD, 0), _part(FD, 1), _rows(FD), _full((FD, FD)), _rows(FD)],
    out_specs=[_rows(FD), _rows(FD)],
    out_shape=[jax.ShapeDtypeStruct((N, FD), jnp.float32)] * 2,
)


def _bridge_body(pa_ref, pb_ref, d_ref, w_ref, yin_ref, b_ref, wn_ref,
                 y_ref, u_ref):
    d = d_ref[...]
    h = d * (pa_ref[0] + pb_ref[0])
    a = jnp.maximum(
        yin_ref[...]
        + jnp.dot(h, w_ref[...], preferred_element_type=jnp.float32)
        + b_ref[...], 0.0)
    y_ref[...] = jnp.dot(a, wn_ref[...], preferred_element_type=jnp.float32)
    u_ref[...] = d * a


_tc_bridge = pl.pallas_call(
    _bridge_body,
    grid=(GRID,),
    in_specs=[_part(FD, 0), _part(FD, 1), _rows(FD), _full((FD, FD)), _rows(FD),
              _full((1, FD)), _full((FD, FD))],
    out_specs=[_rows(FD), _rows(FD)],
    out_shape=[jax.ShapeDtypeStruct((N, FD), jnp.float32)] * 2,
)


def _final_body(pa_ref, pb_ref, d_ref, w_ref, yin_ref, b_ref, wc_ref, bc_ref,
                o_ref):
    d = d_ref[...]
    h = d * (pa_ref[0] + pb_ref[0])
    a = jnp.maximum(
        yin_ref[...]
        + jnp.dot(h, w_ref[...], preferred_element_type=jnp.float32)
        + b_ref[...], 0.0)
    o_ref[...] = jnp.dot(
        a, wc_ref[...], preferred_element_type=jnp.float32) + bc_ref[...]


_tc_final = pl.pallas_call(
    _final_body,
    grid=(GRID,),
    in_specs=[_part(FD, 0), _part(FD, 1), _rows(FD), _full((FD, FD)), _rows(FD),
              _full((1, FD)), _full((FD, NCLS)), _full((1, NCLS))],
    out_specs=_rows(NCLS),
    out_shape=jax.ShapeDtypeStruct((N, NCLS), jnp.float32),
)


def kernel(x, edge_index, W1, b1, W2, b2, Wc, bc):
    ei = edge_index.astype(jnp.int32)
    src = ei[0]
    dst = ei[1]

    dst_deg = dst.reshape(NW, NCHD, BD)
    pad = NCH * B - EW
    src3 = jnp.pad(src.reshape(NW, EW), ((0, 0), (0, pad))).reshape(NW, NCH, B)
    dst3 = jnp.pad(dst.reshape(NW, EW), ((0, 0), (0, pad)),
                   constant_values=DUMP).reshape(NW, NCH, B)

    degp = _sc_degree(dst_deg)
    y, u, dinvb = _tc_prep(x, degp, degp, W1[0])

    for k in (1, 2):
        p = _sc_propagate(u, src3, dst3)
        y, u = _tc_step(p, p, dinvb, W1[k], y)
    p = _sc_propagate(u, src3, dst3)
    y, u = _tc_bridge(p, p, dinvb, W1[3], y, b1.reshape(1, FD), W2[0])

    for k in (1, 2):
        p = _sc_propagate(u, src3, dst3)
        y, u = _tc_step(p, p, dinvb, W2[k], y)
    p = _sc_propagate(u, src3, dst3)
    return _tc_final(p, p, dinvb, W2[3], y, b2.reshape(1, FD),
                     Wc, bc.reshape(1, NCLS))


# speedup vs baseline: 2.3129x; 2.3129x over previous
import functools

import jax
import jax.numpy as jnp
from jax import lax
from jax.experimental import pallas as pl
from jax.experimental.pallas import tpu as pltpu
from jax.experimental.pallas import tpu_sc as plsc

N = 10000
FD = 128
EDGES = 320000
NCLS = 40

NC = 2
NS = 16
NW = NC * NS
NP = 10240
RT = NP // NS
DUMP = 10200

B = 80
EW = EDGES // NW
NCH = EW // B
NPAIR = (NCH - 1) // 2

BD = 100
NCHD = EW // BD
ZR = 128

_MESH = plsc.VectorSubcoreMesh(core_axis_name="c", subcore_axis_name="s")


@functools.partial(
    pl.kernel,
    out_type=jax.ShapeDtypeStruct((2, NP, FD), jnp.float32),
    mesh=_MESH,
    scratch_types=[
        pltpu.VMEM((NCHD, BD), jnp.int32),
        pltpu.VMEM((BD, FD), jnp.float32),
        pltpu.VMEM((ZR, FD), jnp.float32),
        pltpu.VMEM_SHARED((NP, FD), jnp.float32),
    ],
)
def _sc_degree(dst_hbm, out_hbm, didx_v, ones_v, zb_v, deg_sh):
    c = lax.axis_index("c")
    s = lax.axis_index("s")
    wid = s * NC + c
    pltpu.sync_copy(dst_hbm.at[wid], didx_v)

    def _fill_ones(i, _):
        for j in range(FD // 16):
            ones_v[i, pl.ds(16 * j, 16)] = jnp.ones((16,), jnp.float32)
        return 0

    def _fill_zero(i, _):
        for j in range(FD // 16):
            zb_v[i, pl.ds(16 * j, 16)] = jnp.zeros((16,), jnp.float32)
        return 0

    lax.fori_loop(0, BD, _fill_ones, 0)
    lax.fori_loop(0, ZR, _fill_zero, 0)

    r0 = s * RT
    for k in range(RT // ZR):
        pltpu.sync_copy(zb_v, deg_sh.at[pl.ds(r0 + k * ZR, ZR)])
    plsc.subcore_barrier()

    def _chunk(i, _):
        pltpu.sync_copy(ones_v, deg_sh.at[didx_v.at[i]], add=True)
        return 0

    lax.fori_loop(0, NCHD, _chunk, 0)
    plsc.subcore_barrier()
    pltpu.sync_copy(deg_sh.at[pl.ds(r0, RT)], out_hbm.at[c, pl.ds(r0, RT)])


@functools.partial(
    pl.kernel,
    out_type=jax.ShapeDtypeStruct((2, NP, FD), jnp.float32),
    mesh=_MESH,
    scratch_types=[
        [pltpu.VMEM((B,), jnp.int32)] * 2,
        [pltpu.VMEM((B,), jnp.int32)] * 2,
        [pltpu.VMEM((B, FD), jnp.float32)] * 2,
        pltpu.VMEM_SHARED((NP, FD), jnp.float32),
        [pltpu.SemaphoreType.DMA] * 2,
    ],
)
def _sc_propagate(u_hbm, src_hbm, dst_hbm, out_hbm,
                  sidx, didx, rows, acc_sh, semg):
    c = lax.axis_index("c")
    s = lax.axis_index("s")
    wid = s * NC + c
    base = wid * EW

    def _zfill(i, _):
        for j in range(FD // 16):
            rows[0][i, pl.ds(16 * j, 16)] = jnp.zeros((16,), jnp.float32)
        return 0

    lax.fori_loop(0, B, _zfill, 0)

    r0 = s * RT
    for k in range(RT // B):
        pltpu.sync_copy(rows[0], acc_sh.at[pl.ds(r0 + k * B, B)])
    plsc.subcore_barrier()

    def _fetch(i, slot):
        off = pl.multiple_of(base + i * B, 8)
        pltpu.sync_copy(src_hbm.at[pl.ds(off, B)], sidx[slot])
        pltpu.sync_copy(dst_hbm.at[pl.ds(off, B)], didx[slot])
        pltpu.async_copy(u_hbm.at[sidx[slot]], rows[slot], semg[slot])

    def _drain(slot):
        pltpu.make_async_copy(
            u_hbm.at[sidx[slot]], rows[slot], semg[slot]).wait()
        pltpu.sync_copy(rows[slot], acc_sh.at[didx[slot]], add=True)

    _fetch(0, 0)

    def _pair(g, _):
        i0 = 2 * g
        _fetch(i0 + 1, 1)
        _drain(0)

        @pl.when(i0 + 2 < NCH)
        def _():
            _fetch(i0 + 2, 0)
        _drain(1)
        return 0

    lax.fori_loop(0, NPAIR, _pair, 0)
    _drain(0)
    plsc.subcore_barrier()
    pltpu.sync_copy(acc_sh.at[pl.ds(r0, RT)], out_hbm.at[c, pl.ds(r0, RT)])


R = 2000
GRID = N // R


def _rows(width):
    return pl.BlockSpec((R, width), lambda i: (i, 0))


def _part(width, which):
    return pl.BlockSpec((1, R, width), lambda i, w=which: (w, i, 0))


def _full(shape):
    return pl.BlockSpec(shape, lambda i: (0,) * len(shape))


def _prep_body(x_ref, dega_ref, degb_ref, w_ref, y_ref, u_ref, d_ref):
    deg = dega_ref[0, :, 0:1] + degb_ref[0, :, 0:1]
    dinv = jnp.where(deg > 0.0, lax.rsqrt(jnp.maximum(deg, 1e-12)), 0.0)
    dinvb = jnp.broadcast_to(dinv, (R, FD))
    x = x_ref[...]
    y_ref[...] = jnp.dot(x, w_ref[...], preferred_element_type=jnp.float32)
    u_ref[...] = dinvb * x
    d_ref[...] = dinvb


_tc_prep = pl.pallas_call(
    _prep_body,
    grid=(GRID,),
    in_specs=[_rows(FD), _part(FD, 0), _part(FD, 1), _full((FD, FD))],
    out_specs=[_rows(FD), _rows(FD), _rows(FD)],
    out_shape=[jax.ShapeDtypeStruct((N, FD), jnp.float32)] * 3,
)


def _step_body(pa_ref, pb_ref, d_ref, w_ref, yin_ref, y_ref, u_ref):
    d = d_ref[...]
    h = d * (pa_ref[0] + pb_ref[0])
    y_ref[...] = yin_ref[...] + jnp.dot(
        h, w_ref[...], preferred_element_type=jnp.float32)
    u_ref[...] = d * h


_tc_step = pl.pallas_call(
    _step_body,
    grid=(GRID,),
    in_specs=[_part(FD, 0), _part(FD, 1), _rows(FD), _full((FD, FD)), _rows(FD)],
    out_specs=[_rows(FD), _rows(FD)],
    out_shape=[jax.ShapeDtypeStruct((N, FD), jnp.float32)] * 2,
)


def _bridge_body(pa_ref, pb_ref, d_ref, w_ref, yin_ref, b_ref, wn_ref,
                 y_ref, u_ref):
    d = d_ref[...]
    h = d * (pa_ref[0] + pb_ref[0])
    a = jnp.maximum(
        yin_ref[...]
        + jnp.dot(h, w_ref[...], preferred_element_type=jnp.float32)
        + b_ref[...], 0.0)
    y_ref[...] = jnp.dot(a, wn_ref[...], preferred_element_type=jnp.float32)
    u_ref[...] = d * a


_tc_bridge = pl.pallas_call(
    _bridge_body,
    grid=(GRID,),
    in_specs=[_part(FD, 0), _part(FD, 1), _rows(FD), _full((FD, FD)), _rows(FD),
              _full((1, FD)), _full((FD, FD))],
    out_specs=[_rows(FD), _rows(FD)],
    out_shape=[jax.ShapeDtypeStruct((N, FD), jnp.float32)] * 2,
)


def _final_body(pa_ref, pb_ref, d_ref, w_ref, yin_ref, b_ref, wc_ref, bc_ref,
                o_ref):
    d = d_ref[...]
    h = d * (pa_ref[0] + pb_ref[0])
    a = jnp.maximum(
        yin_ref[...]
        + jnp.dot(h, w_ref[...], preferred_element_type=jnp.float32)
        + b_ref[...], 0.0)
    o_ref[...] = jnp.dot(
        a, wc_ref[...], preferred_element_type=jnp.float32) + bc_ref[...]


_tc_final = pl.pallas_call(
    _final_body,
    grid=(GRID,),
    in_specs=[_part(FD, 0), _part(FD, 1), _rows(FD), _full((FD, FD)), _rows(FD),
              _full((1, FD)), _full((FD, NCLS)), _full((1, NCLS))],
    out_specs=_rows(NCLS),
    out_shape=jax.ShapeDtypeStruct((N, NCLS), jnp.float32),
)


def kernel(x, edge_index, W1, b1, W2, b2, Wc, bc):
    ei = edge_index.astype(jnp.int32)
    src = ei[0]
    dst = ei[1]

    dst_deg = dst.reshape(NW, NCHD, BD)

    degp = _sc_degree(dst_deg)
    y, u, dinvb = _tc_prep(x, degp, degp, W1[0])

    for k in (1, 2):
        p = _sc_propagate(u, src, dst)
        y, u = _tc_step(p, p, dinvb, W1[k], y)
    p = _sc_propagate(u, src, dst)
    y, u = _tc_bridge(p, p, dinvb, W1[3], y, b1.reshape(1, FD), W2[0])

    for k in (1, 2):
        p = _sc_propagate(u, src, dst)
        y, u = _tc_step(p, p, dinvb, W2[k], y)
    p = _sc_propagate(u, src, dst)
    return _tc_final(p, p, dinvb, W2[3], y, b2.reshape(1, FD),
                     Wc, bc.reshape(1, NCLS))


# speedup vs baseline: 2.3153x; 1.0010x over previous
import functools

import jax
import jax.numpy as jnp
from jax import lax
from jax.experimental import pallas as pl
from jax.experimental.pallas import tpu as pltpu
from jax.experimental.pallas import tpu_sc as plsc

N = 10000
FD = 128
EDGES = 320000
NCLS = 40

NC = 2
NS = 16
NW = NC * NS
NP = 10240
RT = NP // NS
DUMP = 10200

B = 80
EW = EDGES // NW
NCH = EW // B
NSLOT = 4
NGRP = (NCH - 1) // NSLOT

BD = 100
NCHD = EW // BD
ZR = 128

_MESH = plsc.VectorSubcoreMesh(core_axis_name="c", subcore_axis_name="s")


@functools.partial(
    pl.kernel,
    out_type=jax.ShapeDtypeStruct((2, NP, FD), jnp.float32),
    mesh=_MESH,
    scratch_types=[
        pltpu.VMEM((NCHD, BD), jnp.int32),
        pltpu.VMEM((BD, FD), jnp.float32),
        pltpu.VMEM((ZR, FD), jnp.float32),
        pltpu.VMEM_SHARED((NP, FD), jnp.float32),
    ],
)
def _sc_degree(dst_hbm, out_hbm, didx_v, ones_v, zb_v, deg_sh):
    c = lax.axis_index("c")
    s = lax.axis_index("s")
    wid = s * NC + c
    pltpu.sync_copy(dst_hbm.at[wid], didx_v)

    def _fill_ones(i, _):
        for j in range(FD // 16):
            ones_v[i, pl.ds(16 * j, 16)] = jnp.ones((16,), jnp.float32)
        return 0

    def _fill_zero(i, _):
        for j in range(FD // 16):
            zb_v[i, pl.ds(16 * j, 16)] = jnp.zeros((16,), jnp.float32)
        return 0

    lax.fori_loop(0, BD, _fill_ones, 0)
    lax.fori_loop(0, ZR, _fill_zero, 0)

    r0 = s * RT
    for k in range(RT // ZR):
        pltpu.sync_copy(zb_v, deg_sh.at[pl.ds(r0 + k * ZR, ZR)])
    plsc.subcore_barrier()

    def _chunk(i, _):
        pltpu.sync_copy(ones_v, deg_sh.at[didx_v.at[i]], add=True)
        return 0

    lax.fori_loop(0, NCHD, _chunk, 0)
    plsc.subcore_barrier()
    pltpu.sync_copy(deg_sh.at[pl.ds(r0, RT)], out_hbm.at[c, pl.ds(r0, RT)])


@functools.partial(
    pl.kernel,
    out_type=jax.ShapeDtypeStruct((2, NP, FD), jnp.float32),
    mesh=_MESH,
    scratch_types=[
        [pltpu.VMEM((B,), jnp.int32)] * NSLOT,
        [pltpu.VMEM((B,), jnp.int32)] * NSLOT,
        [pltpu.VMEM((B, FD), jnp.float32)] * NSLOT,
        pltpu.VMEM_SHARED((NP, FD), jnp.float32),
        [pltpu.SemaphoreType.DMA] * NSLOT,
    ],
)
def _sc_propagate(u_hbm, src_hbm, dst_hbm, out_hbm,
                  sidx, didx, rows, acc_sh, semg):
    c = lax.axis_index("c")
    s = lax.axis_index("s")
    wid = s * NC + c
    base = wid * EW

    def _zfill(i, _):
        for j in range(FD // 16):
            rows[0][i, pl.ds(16 * j, 16)] = jnp.zeros((16,), jnp.float32)
        return 0

    lax.fori_loop(0, B, _zfill, 0)

    r0 = s * RT
    for k in range(RT // B):
        pltpu.sync_copy(rows[0], acc_sh.at[pl.ds(r0 + k * B, B)])
    plsc.subcore_barrier()

    def _fetch(i, slot):
        off = pl.multiple_of(base + i * B, 8)
        pltpu.sync_copy(src_hbm.at[pl.ds(off, B)], sidx[slot])
        pltpu.sync_copy(dst_hbm.at[pl.ds(off, B)], didx[slot])
        pltpu.async_copy(u_hbm.at[sidx[slot]], rows[slot], semg[slot])

    def _drain(slot):
        pltpu.make_async_copy(
            u_hbm.at[sidx[slot]], rows[slot], semg[slot]).wait()
        pltpu.sync_copy(rows[slot], acc_sh.at[didx[slot]], add=True)

    for q in range(NSLOT - 1):
        _fetch(q, q)

    def _group(g, _):
        for q in range(NSLOT):
            i = g * NSLOT + q

            @pl.when(i + NSLOT - 1 < NCH)
            def _():
                _fetch(i + NSLOT - 1, (q + NSLOT - 1) % NSLOT)
            _drain(q)
        return 0

    lax.fori_loop(0, NGRP, _group, 0)
    _drain(0)
    plsc.subcore_barrier()
    pltpu.sync_copy(acc_sh.at[pl.ds(r0, RT)], out_hbm.at[c, pl.ds(r0, RT)])


R = 2000
GRID = N // R


def _rows(width):
    return pl.BlockSpec((R, width), lambda i: (i, 0))


def _part(width, which):
    return pl.BlockSpec((1, R, width), lambda i, w=which: (w, i, 0))


def _full(shape):
    return pl.BlockSpec(shape, lambda i: (0,) * len(shape))


def _prep_body(x_ref, dega_ref, degb_ref, w_ref, y_ref, u_ref, d_ref):
    deg = dega_ref[0, :, 0:1] + degb_ref[0, :, 0:1]
    dinv = jnp.where(deg > 0.0, lax.rsqrt(jnp.maximum(deg, 1e-12)), 0.0)
    dinvb = jnp.broadcast_to(dinv, (R, FD))
    x = x_ref[...]
    y_ref[...] = jnp.dot(x, w_ref[...], preferred_element_type=jnp.float32)
    u_ref[...] = dinvb * x
    d_ref[...] = dinvb


_tc_prep = pl.pallas_call(
    _prep_body,
    grid=(GRID,),
    in_specs=[_rows(FD), _part(FD, 0), _part(FD, 1), _full((FD, FD))],
    out_specs=[_rows(FD), _rows(FD), _rows(FD)],
    out_shape=[jax.ShapeDtypeStruct((N, FD), jnp.float32)] * 3,
)


def _step_body(pa_ref, pb_ref, d_ref, w_ref, yin_ref, y_ref, u_ref):
    d = d_ref[...]
    h = d * (pa_ref[0] + pb_ref[0])
    y_ref[...] = yin_ref[...] + jnp.dot(
        h, w_ref[...], preferred_element_type=jnp.float32)
    u_ref[...] = d * h


_tc_step = pl.pallas_call(
    _step_body,
    grid=(GRID,),
    in_specs=[_part(FD, 0), _part(FD, 1), _rows(FD), _full((FD, FD)), _rows(FD)],
    out_specs=[_rows(FD), _rows(FD)],
    out_shape=[jax.ShapeDtypeStruct((N, FD), jnp.float32)] * 2,
)


def _bridge_body(pa_ref, pb_ref, d_ref, w_ref, yin_ref, b_ref, wn_ref,
                 y_ref, u_ref):
    d = d_ref[...]
    h = d * (pa_ref[0] + pb_ref[0])
    a = jnp.maximum(
        yin_ref[...]
        + jnp.dot(h, w_ref[...], preferred_element_type=jnp.float32)
        + b_ref[...], 0.0)
    y_ref[...] = jnp.dot(a, wn_ref[...], preferred_element_type=jnp.float32)
    u_ref[...] = d * a


_tc_bridge = pl.pallas_call(
    _bridge_body,
    grid=(GRID,),
    in_specs=[_part(FD, 0), _part(FD, 1), _rows(FD), _full((FD, FD)), _rows(FD),
              _full((1, FD)), _full((FD, FD))],
    out_specs=[_rows(FD), _rows(FD)],
    out_shape=[jax.ShapeDtypeStruct((N, FD), jnp.float32)] * 2,
)


def _final_body(pa_ref, pb_ref, d_ref, w_ref, yin_ref, b_ref, wc_ref, bc_ref,
                o_ref):
    d = d_ref[...]
    h = d * (pa_ref[0] + pb_ref[0])
    a = jnp.maximum(
        yin_ref[...]
        + jnp.dot(h, w_ref[...], preferred_element_type=jnp.float32)
        + b_ref[...], 0.0)
    o_ref[...] = jnp.dot(
        a, wc_ref[...], preferred_element_type=jnp.float32) + bc_ref[...]


_tc_final = pl.pallas_call(
    _final_body,
    grid=(GRID,),
    in_specs=[_part(FD, 0), _part(FD, 1), _rows(FD), _full((FD, FD)), _rows(FD),
              _full((1, FD)), _full((FD, NCLS)), _full((1, NCLS))],
    out_specs=_rows(NCLS),
    out_shape=jax.ShapeDtypeStruct((N, NCLS), jnp.float32),
)


def kernel(x, edge_index, W1, b1, W2, b2, Wc, bc):
    ei = edge_index.astype(jnp.int32)
    src = ei[0]
    dst = ei[1]

    dst_deg = dst.reshape(NW, NCHD, BD)

    degp = _sc_degree(dst_deg)
    y, u, dinvb = _tc_prep(x, degp, degp, W1[0])

    for k in (1, 2):
        p = _sc_propagate(u, src, dst)
        y, u = _tc_step(p, p, dinvb, W1[k], y)
    p = _sc_propagate(u, src, dst)
    y, u = _tc_bridge(p, p, dinvb, W1[3], y, b1.reshape(1, FD), W2[0])

    for k in (1, 2):
        p = _sc_propagate(u, src, dst)
        y, u = _tc_step(p, p, dinvb, W2[k], y)
    p = _sc_propagate(u, src, dst)
    return _tc_final(p, p, dinvb, W2[3], y, b2.reshape(1, FD),
                     Wc, bc.reshape(1, NCLS))


# speedup vs baseline: 2.3197x; 1.0019x over previous
import functools

import jax
import jax.numpy as jnp
from jax import lax
from jax.experimental import pallas as pl
from jax.experimental.pallas import tpu as pltpu
from jax.experimental.pallas import tpu_sc as plsc

N = 10000
FD = 128
EDGES = 320000
NCLS = 40

NC = 2
NS = 16
NW = NC * NS
NP = 10240
RT = NP // NS
DUMP = 10200

B = 80
EW = EDGES // NW
NCH = EW // B
NSLOT = 4
NGRP = (NCH - 1) // NSLOT

BD = 100
NCHD = EW // BD
ZR = 128

_MESH = plsc.VectorSubcoreMesh(core_axis_name="c", subcore_axis_name="s")


@functools.partial(
    pl.kernel,
    out_type=jax.ShapeDtypeStruct((2, NP, FD), jnp.float32),
    mesh=_MESH,
    scratch_types=[
        pltpu.VMEM((NCHD, BD), jnp.int32),
        pltpu.VMEM((BD, FD), jnp.float32),
        pltpu.VMEM((ZR, FD), jnp.float32),
        pltpu.VMEM_SHARED((NP, FD), jnp.float32),
    ],
)
def _sc_degree(dst_hbm, out_hbm, didx_v, ones_v, zb_v, deg_sh):
    c = lax.axis_index("c")
    s = lax.axis_index("s")
    wid = s * NC + c
    pltpu.sync_copy(dst_hbm.at[wid], didx_v)

    def _fill_ones(i, _):
        for j in range(FD // 16):
            ones_v[i, pl.ds(16 * j, 16)] = jnp.ones((16,), jnp.float32)
        return 0

    def _fill_zero(i, _):
        for j in range(FD // 16):
            zb_v[i, pl.ds(16 * j, 16)] = jnp.zeros((16,), jnp.float32)
        return 0

    lax.fori_loop(0, BD, _fill_ones, 0)
    lax.fori_loop(0, ZR, _fill_zero, 0)

    r0 = s * RT
    for k in range(RT // ZR):
        pltpu.sync_copy(zb_v, deg_sh.at[pl.ds(r0 + k * ZR, ZR)])
    plsc.subcore_barrier()

    def _chunk(i, _):
        pltpu.sync_copy(ones_v, deg_sh.at[didx_v.at[i]], add=True)
        return 0

    lax.fori_loop(0, NCHD, _chunk, 0)
    plsc.subcore_barrier()
    pltpu.sync_copy(deg_sh.at[pl.ds(r0, RT)], out_hbm.at[c, pl.ds(r0, RT)])


@functools.partial(
    pl.kernel,
    out_type=jax.ShapeDtypeStruct((2, NP, FD), jnp.float32),
    mesh=_MESH,
    scratch_types=[
        [pltpu.VMEM((B,), jnp.int32)] * NSLOT,
        [pltpu.VMEM((B,), jnp.int32)] * NSLOT,
        [pltpu.VMEM((B, FD), jnp.float32)] * NSLOT,
        pltpu.VMEM_SHARED((NP, FD), jnp.float32),
        [pltpu.SemaphoreType.DMA] * NSLOT,
        [pltpu.SemaphoreType.DMA] * NSLOT,
    ],
)
def _sc_propagate(u_hbm, src_hbm, dst_hbm, out_hbm,
                  sidx, didx, rows, acc_sh, semg, sems):
    c = lax.axis_index("c")
    s = lax.axis_index("s")
    wid = s * NC + c
    base = wid * EW

    def _zfill(i, _):
        for j in range(FD // 16):
            rows[0][i, pl.ds(16 * j, 16)] = jnp.zeros((16,), jnp.float32)
        return 0

    lax.fori_loop(0, B, _zfill, 0)

    r0 = s * RT
    for k in range(RT // B):
        pltpu.sync_copy(rows[0], acc_sh.at[pl.ds(r0 + k * B, B)])
    plsc.subcore_barrier()

    def _fetch(i, slot):
        off = pl.multiple_of(base + i * B, 8)
        pltpu.sync_copy(src_hbm.at[pl.ds(off, B)], sidx[slot])
        pltpu.sync_copy(dst_hbm.at[pl.ds(off, B)], didx[slot])
        pltpu.async_copy(u_hbm.at[sidx[slot]], rows[slot], semg[slot])

    def _wait_scatter(slot):
        pltpu.make_async_copy(
            rows[slot], acc_sh.at[didx[slot]], sems[slot]).wait()

    def _drain(slot):
        pltpu.make_async_copy(
            u_hbm.at[sidx[slot]], rows[slot], semg[slot]).wait()
        pltpu.async_copy(rows[slot], acc_sh.at[didx[slot]], sems[slot],
                         add=True)

    for q in range(NSLOT - 1):
        _fetch(q, q)

    def _group(g, _):
        for q in range(NSLOT):
            i = g * NSLOT + q

            @pl.when(i + NSLOT - 1 < NCH)
            def _():
                slot = (q + NSLOT - 1) % NSLOT

                @pl.when(i + NSLOT - 1 >= NSLOT)
                def _():
                    _wait_scatter(slot)
                _fetch(i + NSLOT - 1, slot)
            _drain(q)
        return 0

    lax.fori_loop(0, NGRP, _group, 0)
    _drain(0)
    for q in range(NSLOT):
        _wait_scatter(q)
    plsc.subcore_barrier()
    pltpu.sync_copy(acc_sh.at[pl.ds(r0, RT)], out_hbm.at[c, pl.ds(r0, RT)])


R = 2000
GRID = N // R


def _rows(width):
    return pl.BlockSpec((R, width), lambda i: (i, 0))


def _part(width, which):
    return pl.BlockSpec((1, R, width), lambda i, w=which: (w, i, 0))


def _full(shape):
    return pl.BlockSpec(shape, lambda i: (0,) * len(shape))


def _prep_body(x_ref, dega_ref, degb_ref, w_ref, y_ref, u_ref, d_ref):
    deg = dega_ref[0, :, 0:1] + degb_ref[0, :, 0:1]
    dinv = jnp.where(deg > 0.0, lax.rsqrt(jnp.maximum(deg, 1e-12)), 0.0)
    dinvb = jnp.broadcast_to(dinv, (R, FD))
    x = x_ref[...]
    y_ref[...] = jnp.dot(x, w_ref[...], preferred_element_type=jnp.float32)
    u_ref[...] = dinvb * x
    d_ref[...] = dinvb


_tc_prep = pl.pallas_call(
    _prep_body,
    grid=(GRID,),
    in_specs=[_rows(FD), _part(FD, 0), _part(FD, 1), _full((FD, FD))],
    out_specs=[_rows(FD), _rows(FD), _rows(FD)],
    out_shape=[jax.ShapeDtypeStruct((N, FD), jnp.float32)] * 3,
)


def _step_body(pa_ref, pb_ref, d_ref, w_ref, yin_ref, y_ref, u_ref):
    d = d_ref[...]
    h = d * (pa_ref[0] + pb_ref[0])
    y_ref[...] = yin_ref[...] + jnp.dot(
        h, w_ref[...], preferred_element_type=jnp.float32)
    u_ref[...] = d * h


_tc_step = pl.pallas_call(
    _step_body,
    grid=(GRID,),
    in_specs=[_part(FD, 0), _part(FD, 1), _rows(FD), _full((FD, FD)), _rows(FD)],
    out_specs=[_rows(FD), _rows(FD)],
    out_shape=[jax.ShapeDtypeStruct((N, FD), jnp.float32)] * 2,
)


def _bridge_body(pa_ref, pb_ref, d_ref, w_ref, yin_ref, b_ref, wn_ref,
                 y_ref, u_ref):
    d = d_ref[...]
    h = d * (pa_ref[0] + pb_ref[0])
    a = jnp.maximum(
        yin_ref[...]
        + jnp.dot(h, w_ref[...], preferred_element_type=jnp.float32)
        + b_ref[...], 0.0)
    y_ref[...] = jnp.dot(a, wn_ref[...], preferred_element_type=jnp.float32)
    u_ref[...] = d * a


_tc_bridge = pl.pallas_call(
    _bridge_body,
    grid=(GRID,),
    in_specs=[_part(FD, 0), _part(FD, 1), _rows(FD), _full((FD, FD)), _rows(FD),
              _full((1, FD)), _full((FD, FD))],
    out_specs=[_rows(FD), _rows(FD)],
    out_shape=[jax.ShapeDtypeStruct((N, FD), jnp.float32)] * 2,
)


def _final_body(pa_ref, pb_ref, d_ref, w_ref, yin_ref, b_ref, wc_ref, bc_ref,
                o_ref):
    d = d_ref[...]
    h = d * (pa_ref[0] + pb_ref[0])
    a = jnp.maximum(
        yin_ref[...]
        + jnp.dot(h, w_ref[...], preferred_element_type=jnp.float32)
        + b_ref[...], 0.0)
    o_ref[...] = jnp.dot(
        a, wc_ref[...], preferred_element_type=jnp.float32) + bc_ref[...]


_tc_final = pl.pallas_call(
    _final_body,
    grid=(GRID,),
    in_specs=[_part(FD, 0), _part(FD, 1), _rows(FD), _full((FD, FD)), _rows(FD),
              _full((1, FD)), _full((FD, NCLS)), _full((1, NCLS))],
    out_specs=_rows(NCLS),
    out_shape=jax.ShapeDtypeStruct((N, NCLS), jnp.float32),
)


def kernel(x, edge_index, W1, b1, W2, b2, Wc, bc):
    ei = edge_index.astype(jnp.int32)
    src = ei[0]
    dst = ei[1]

    dst_deg = dst.reshape(NW, NCHD, BD)

    degp = _sc_degree(dst_deg)
    y, u, dinvb = _tc_prep(x, degp, degp, W1[0])

    for k in (1, 2):
        p = _sc_propagate(u, src, dst)
        y, u = _tc_step(p, p, dinvb, W1[k], y)
    p = _sc_propagate(u, src, dst)
    y, u = _tc_bridge(p, p, dinvb, W1[3], y, b1.reshape(1, FD), W2[0])

    for k in (1, 2):
        p = _sc_propagate(u, src, dst)
        y, u = _tc_step(p, p, dinvb, W2[k], y)
    p = _sc_propagate(u, src, dst)
    return _tc_final(p, p, dinvb, W2[3], y, b2.reshape(1, FD),
                     Wc, bc.reshape(1, NCLS))


# speedup vs baseline: 3.9510x; 1.7033x over previous
import functools

import jax
import jax.numpy as jnp
from jax import lax
from jax.experimental import pallas as pl
from jax.experimental.pallas import tpu as pltpu
from jax.experimental.pallas import tpu_sc as plsc

N = 10000
FD = 128
EDGES = 320000
NCLS = 40

NC = 2
NS = 16
NW = NC * NS
NP = 10240
RT = NP // NS
DUMP = 10200

B = 80
EW = EDGES // NW
NCH = EW // B
NSLOT = 4
NGRP = (NCH - 1) // NSLOT

BD = 100
NCHD = EW // BD
ZR = 128

_MESH = plsc.VectorSubcoreMesh(core_axis_name="c", subcore_axis_name="s")


@functools.partial(
    pl.kernel,
    out_type=jax.ShapeDtypeStruct((2, NP, FD), jnp.float32),
    mesh=_MESH,
    scratch_types=[
        pltpu.VMEM((NCHD, BD), jnp.int32),
        pltpu.VMEM((BD, FD), jnp.float32),
        pltpu.VMEM((ZR, FD), jnp.float32),
        pltpu.VMEM_SHARED((NP, FD), jnp.float32),
    ],
)
def _sc_degree(dst_hbm, out_hbm, didx_v, ones_v, zb_v, deg_sh):
    c = lax.axis_index("c")
    s = lax.axis_index("s")
    wid = s * NC + c
    pltpu.sync_copy(dst_hbm.at[wid], didx_v)

    def _fill_ones(i, _):
        for j in range(FD // 16):
            ones_v[i, pl.ds(16 * j, 16)] = jnp.ones((16,), jnp.float32)
        return 0

    def _fill_zero(i, _):
        for j in range(FD // 16):
            zb_v[i, pl.ds(16 * j, 16)] = jnp.zeros((16,), jnp.float32)
        return 0

    lax.fori_loop(0, BD, _fill_ones, 0)
    lax.fori_loop(0, ZR, _fill_zero, 0)

    r0 = s * RT
    for k in range(RT // ZR):
        pltpu.sync_copy(zb_v, deg_sh.at[pl.ds(r0 + k * ZR, ZR)])
    plsc.subcore_barrier()

    def _chunk(i, _):
        pltpu.sync_copy(ones_v, deg_sh.at[didx_v.at[i]], add=True)
        return 0

    lax.fori_loop(0, NCHD, _chunk, 0)
    plsc.subcore_barrier()
    pltpu.sync_copy(deg_sh.at[pl.ds(r0, RT)], out_hbm.at[c, pl.ds(r0, RT)])


@functools.partial(
    pl.kernel,
    out_type=jax.ShapeDtypeStruct((2, NP, FD), jnp.float32),
    mesh=_MESH,
    scratch_types=[
        [pltpu.VMEM((B,), jnp.int32)] * NSLOT,
        [pltpu.VMEM((B,), jnp.int32)] * NSLOT,
        [pltpu.VMEM((B, FD), jnp.float32)] * NSLOT,
        pltpu.VMEM_SHARED((NP, FD), jnp.float32),
        [pltpu.SemaphoreType.DMA] * NSLOT,
        [pltpu.SemaphoreType.DMA] * NSLOT,
        [pltpu.SemaphoreType.DMA] * NSLOT,
    ],
)
def _sc_propagate(u_hbm, src_hbm, dst_hbm, out_hbm,
                  sidx, didx, rows, acc_sh, semg, sems, semi):
    c = lax.axis_index("c")
    s = lax.axis_index("s")
    wid = s * NC + c
    base = wid * EW

    def _zfill(i, _):
        for j in range(FD // 16):
            rows[0][i, pl.ds(16 * j, 16)] = jnp.zeros((16,), jnp.float32)
        return 0

    lax.fori_loop(0, B, _zfill, 0)

    r0 = s * RT
    for k in range(RT // B):
        pltpu.sync_copy(rows[0], acc_sh.at[pl.ds(r0 + k * B, B)])
    plsc.subcore_barrier()

    def _fetch_idx(i, slot):
        off = pl.multiple_of(base + i * B, 8)
        pltpu.async_copy(src_hbm.at[pl.ds(off, B)], sidx[slot], semi[slot])
        pltpu.async_copy(dst_hbm.at[pl.ds(off, B)], didx[slot], semi[slot])

    def _gather(i, slot):
        off = pl.multiple_of(base + i * B, 8)
        pltpu.make_async_copy(
            src_hbm.at[pl.ds(off, B)], sidx[slot], semi[slot]).wait()
        pltpu.make_async_copy(
            dst_hbm.at[pl.ds(off, B)], didx[slot], semi[slot]).wait()
        pltpu.async_copy(u_hbm.at[sidx[slot]], rows[slot], semg[slot])

    def _wait_scatter(slot):
        pltpu.make_async_copy(
            rows[slot], acc_sh.at[didx[slot]], sems[slot]).wait()

    def _drain(slot):
        pltpu.make_async_copy(
            u_hbm.at[sidx[slot]], rows[slot], semg[slot]).wait()
        pltpu.async_copy(rows[slot], acc_sh.at[didx[slot]], sems[slot],
                         add=True)

    for q in range(NSLOT - 1):
        _fetch_idx(q, q)
    for q in range(NSLOT - 2):
        _gather(q, q)

    def _group(g, _):
        for q in range(NSLOT):
            i = g * NSLOT + q
            sl3 = (q + NSLOT - 1) % NSLOT
            sl2 = (q + NSLOT - 2) % NSLOT

            @pl.when(i + NSLOT - 1 < NCH)
            def _():
                @pl.when(i + NSLOT - 1 >= NSLOT)
                def _():
                    _wait_scatter(sl3)
                _fetch_idx(i + NSLOT - 1, sl3)

            @pl.when(i + NSLOT - 2 < NCH)
            def _():
                _gather(i + NSLOT - 2, sl2)
            _drain(q)
        return 0

    lax.fori_loop(0, NGRP, _group, 0)
    _drain(0)
    for q in range(NSLOT):
        _wait_scatter(q)
    plsc.subcore_barrier()
    pltpu.sync_copy(acc_sh.at[pl.ds(r0, RT)], out_hbm.at[c, pl.ds(r0, RT)])


R = 2000
GRID = N // R


def _rows(width):
    return pl.BlockSpec((R, width), lambda i: (i, 0))


def _part(width, which):
    return pl.BlockSpec((1, R, width), lambda i, w=which: (w, i, 0))


def _full(shape):
    return pl.BlockSpec(shape, lambda i: (0,) * len(shape))


def _prep_body(x_ref, dega_ref, degb_ref, w_ref, y_ref, u_ref, d_ref):
    deg = dega_ref[0, :, 0:1] + degb_ref[0, :, 0:1]
    dinv = jnp.where(deg > 0.0, lax.rsqrt(jnp.maximum(deg, 1e-12)), 0.0)
    dinvb = jnp.broadcast_to(dinv, (R, FD))
    x = x_ref[...]
    y_ref[...] = jnp.dot(x, w_ref[...], preferred_element_type=jnp.float32)
    u_ref[...] = dinvb * x
    d_ref[...] = dinvb


_tc_prep = pl.pallas_call(
    _prep_body,
    grid=(GRID,),
    in_specs=[_rows(FD), _part(FD, 0), _part(FD, 1), _full((FD, FD))],
    out_specs=[_rows(FD), _rows(FD), _rows(FD)],
    out_shape=[jax.ShapeDtypeStruct((N, FD), jnp.float32)] * 3,
)


def _step_body(pa_ref, pb_ref, d_ref, w_ref, yin_ref, y_ref, u_ref):
    d = d_ref[...]
    h = d * (pa_ref[0] + pb_ref[0])
    y_ref[...] = yin_ref[...] + jnp.dot(
        h, w_ref[...], preferred_element_type=jnp.float32)
    u_ref[...] = d * h


_tc_step = pl.pallas_call(
    _step_body,
    grid=(GRID,),
    in_specs=[_part(FD, 0), _part(FD, 1), _rows(FD), _full((FD, FD)), _rows(FD)],
    out_specs=[_rows(FD), _rows(FD)],
    out_shape=[jax.ShapeDtypeStruct((N, FD), jnp.float32)] * 2,
)


def _bridge_body(pa_ref, pb_ref, d_ref, w_ref, yin_ref, b_ref, wn_ref,
                 y_ref, u_ref):
    d = d_ref[...]
    h = d * (pa_ref[0] + pb_ref[0])
    a = jnp.maximum(
        yin_ref[...]
        + jnp.dot(h, w_ref[...], preferred_element_type=jnp.float32)
        + b_ref[...], 0.0)
    y_ref[...] = jnp.dot(a, wn_ref[...], preferred_element_type=jnp.float32)
    u_ref[...] = d * a


_tc_bridge = pl.pallas_call(
    _bridge_body,
    grid=(GRID,),
    in_specs=[_part(FD, 0), _part(FD, 1), _rows(FD), _full((FD, FD)), _rows(FD),
              _full((1, FD)), _full((FD, FD))],
    out_specs=[_rows(FD), _rows(FD)],
    out_shape=[jax.ShapeDtypeStruct((N, FD), jnp.float32)] * 2,
)


def _final_body(pa_ref, pb_ref, d_ref, w_ref, yin_ref, b_ref, wc_ref, bc_ref,
                o_ref):
    d = d_ref[...]
    h = d * (pa_ref[0] + pb_ref[0])
    a = jnp.maximum(
        yin_ref[...]
        + jnp.dot(h, w_ref[...], preferred_element_type=jnp.float32)
        + b_ref[...], 0.0)
    o_ref[...] = jnp.dot(
        a, wc_ref[...], preferred_element_type=jnp.float32) + bc_ref[...]


_tc_final = pl.pallas_call(
    _final_body,
    grid=(GRID,),
    in_specs=[_part(FD, 0), _part(FD, 1), _rows(FD), _full((FD, FD)), _rows(FD),
              _full((1, FD)), _full((FD, NCLS)), _full((1, NCLS))],
    out_specs=_rows(NCLS),
    out_shape=jax.ShapeDtypeStruct((N, NCLS), jnp.float32),
)


def kernel(x, edge_index, W1, b1, W2, b2, Wc, bc):
    ei = edge_index.astype(jnp.int32)
    src = ei[0]
    dst = ei[1]

    dst_deg = dst.reshape(NW, NCHD, BD)

    degp = _sc_degree(dst_deg)
    y, u, dinvb = _tc_prep(x, degp, degp, W1[0])

    for k in (1, 2):
        p = _sc_propagate(u, src, dst)
        y, u = _tc_step(p, p, dinvb, W1[k], y)
    p = _sc_propagate(u, src, dst)
    y, u = _tc_bridge(p, p, dinvb, W1[3], y, b1.reshape(1, FD), W2[0])

    for k in (1, 2):
        p = _sc_propagate(u, src, dst)
        y, u = _tc_step(p, p, dinvb, W2[k], y)
    p = _sc_propagate(u, src, dst)
    return _tc_final(p, p, dinvb, W2[3], y, b2.reshape(1, FD),
                     Wc, bc.reshape(1, NCLS))


# speedup vs baseline: 3.9578x; 1.0017x over previous
import functools

import jax
import jax.numpy as jnp
from jax import lax
from jax.experimental import pallas as pl
from jax.experimental.pallas import tpu as pltpu
from jax.experimental.pallas import tpu_sc as plsc

N = 10000
FD = 128
EDGES = 320000
NCLS = 40

NC = 2
NS = 16
NW = NC * NS
NP = 10240
RT = NP // NS
DUMP = 10200

B = 80
EW = EDGES // NW
NCH = EW // B
NSLOT = 4
NGRP = (NCH - 1) // NSLOT

BD = 100
NCHD = EW // BD
ZR = 128

_MESH = plsc.VectorSubcoreMesh(core_axis_name="c", subcore_axis_name="s")


@functools.partial(
    pl.kernel,
    out_type=jax.ShapeDtypeStruct((2, NP, FD), jnp.float32),
    mesh=_MESH,
    scratch_types=[
        pltpu.VMEM((NCHD, BD), jnp.int32),
        pltpu.VMEM((BD, FD), jnp.float32),
        pltpu.VMEM((ZR, FD), jnp.float32),
        pltpu.VMEM_SHARED((NP, FD), jnp.float32),
        [pltpu.SemaphoreType.DMA] * 4,
    ],
)
def _sc_degree(dst_hbm, out_hbm, didx_v, ones_v, zb_v, deg_sh, sems):
    c = lax.axis_index("c")
    s = lax.axis_index("s")
    wid = s * NC + c
    pltpu.sync_copy(dst_hbm.at[wid], didx_v)

    def _fill_ones(i, _):
        for j in range(FD // 16):
            ones_v[i, pl.ds(16 * j, 16)] = jnp.ones((16,), jnp.float32)
        return 0

    def _fill_zero(i, _):
        for j in range(FD // 16):
            zb_v[i, pl.ds(16 * j, 16)] = jnp.zeros((16,), jnp.float32)
        return 0

    lax.fori_loop(0, BD, _fill_ones, 0)
    lax.fori_loop(0, ZR, _fill_zero, 0)

    r0 = s * RT
    for k in range(RT // ZR):
        pltpu.sync_copy(zb_v, deg_sh.at[pl.ds(r0 + k * ZR, ZR)])
    plsc.subcore_barrier()

    def _sc_issue(i, q):
        pltpu.async_copy(ones_v, deg_sh.at[didx_v.at[i]], sems[q], add=True)

    def _sc_wait(i, q):
        pltpu.make_async_copy(ones_v, deg_sh.at[didx_v.at[i]], sems[q]).wait()

    for q in range(4):
        _sc_issue(q, q)

    def _chunk(g, _):
        for q in range(4):
            i = 4 * g + q
            _sc_wait(i, q)

            @pl.when(i + 4 < NCHD)
            def _():
                _sc_issue(i + 4, q)
        return 0

    lax.fori_loop(0, NCHD // 4, _chunk, 0)
    plsc.subcore_barrier()
    pltpu.sync_copy(deg_sh.at[pl.ds(r0, RT)], out_hbm.at[c, pl.ds(r0, RT)])


@functools.partial(
    pl.kernel,
    out_type=jax.ShapeDtypeStruct((2, NP, FD), jnp.float32),
    mesh=_MESH,
    scratch_types=[
        [pltpu.VMEM((B,), jnp.int32)] * NSLOT,
        [pltpu.VMEM((B,), jnp.int32)] * NSLOT,
        [pltpu.VMEM((B, FD), jnp.float32)] * NSLOT,
        pltpu.VMEM_SHARED((NP, FD), jnp.float32),
        [pltpu.SemaphoreType.DMA] * NSLOT,
        [pltpu.SemaphoreType.DMA] * NSLOT,
        [pltpu.SemaphoreType.DMA] * NSLOT,
    ],
)
def _sc_propagate(u_hbm, src_hbm, dst_hbm, out_hbm,
                  sidx, didx, rows, acc_sh, semg, sems, semi):
    c = lax.axis_index("c")
    s = lax.axis_index("s")
    wid = s * NC + c
    base = wid * EW

    def _zfill(i, _):
        for j in range(FD // 16):
            rows[0][i, pl.ds(16 * j, 16)] = jnp.zeros((16,), jnp.float32)
        return 0

    lax.fori_loop(0, B, _zfill, 0)

    r0 = s * RT
    for k in range(RT // B):
        pltpu.sync_copy(rows[0], acc_sh.at[pl.ds(r0 + k * B, B)])
    plsc.subcore_barrier()

    def _fetch_idx(i, slot):
        off = pl.multiple_of(base + i * B, 8)
        pltpu.async_copy(src_hbm.at[pl.ds(off, B)], sidx[slot], semi[slot])
        pltpu.async_copy(dst_hbm.at[pl.ds(off, B)], didx[slot], semi[slot])

    def _gather(i, slot):
        off = pl.multiple_of(base + i * B, 8)
        pltpu.make_async_copy(
            src_hbm.at[pl.ds(off, B)], sidx[slot], semi[slot]).wait()
        pltpu.make_async_copy(
            dst_hbm.at[pl.ds(off, B)], didx[slot], semi[slot]).wait()
        pltpu.async_copy(u_hbm.at[sidx[slot]], rows[slot], semg[slot])

    def _wait_scatter(slot):
        pltpu.make_async_copy(
            rows[slot], acc_sh.at[didx[slot]], sems[slot]).wait()

    def _drain(slot):
        pltpu.make_async_copy(
            u_hbm.at[sidx[slot]], rows[slot], semg[slot]).wait()
        pltpu.async_copy(rows[slot], acc_sh.at[didx[slot]], sems[slot],
                         add=True)

    for q in range(NSLOT - 1):
        _fetch_idx(q, q)
    for q in range(NSLOT - 2):
        _gather(q, q)

    def _group(g, _):
        for q in range(NSLOT):
            i = g * NSLOT + q
            sl3 = (q + NSLOT - 1) % NSLOT
            sl2 = (q + NSLOT - 2) % NSLOT

            @pl.when(i + NSLOT - 1 < NCH)
            def _():
                @pl.when(i + NSLOT - 1 >= NSLOT)
                def _():
                    _wait_scatter(sl3)
                _fetch_idx(i + NSLOT - 1, sl3)

            @pl.when(i + NSLOT - 2 < NCH)
            def _():
                _gather(i + NSLOT - 2, sl2)
            _drain(q)
        return 0

    lax.fori_loop(0, NGRP, _group, 0)
    _drain(0)
    for q in range(NSLOT):
        _wait_scatter(q)
    plsc.subcore_barrier()
    pltpu.sync_copy(acc_sh.at[pl.ds(r0, RT)], out_hbm.at[c, pl.ds(r0, RT)])


R = 2000
GRID = N // R


def _rows(width):
    return pl.BlockSpec((R, width), lambda i: (i, 0))


def _part(width, which):
    return pl.BlockSpec((1, R, width), lambda i, w=which: (w, i, 0))


def _full(shape):
    return pl.BlockSpec(shape, lambda i: (0,) * len(shape))


def _prep_body(x_ref, dega_ref, degb_ref, w_ref, y_ref, u_ref, d_ref):
    deg = dega_ref[0, :, 0:1] + degb_ref[0, :, 0:1]
    dinv = jnp.where(deg > 0.0, lax.rsqrt(jnp.maximum(deg, 1e-12)), 0.0)
    dinvb = jnp.broadcast_to(dinv, (R, FD))
    x = x_ref[...]
    y_ref[...] = jnp.dot(x, w_ref[...], preferred_element_type=jnp.float32)
    u_ref[...] = dinvb * x
    d_ref[...] = dinvb


_tc_prep = pl.pallas_call(
    _prep_body,
    grid=(GRID,),
    in_specs=[_rows(FD), _part(FD, 0), _part(FD, 1), _full((FD, FD))],
    out_specs=[_rows(FD), _rows(FD), _rows(FD)],
    out_shape=[jax.ShapeDtypeStruct((N, FD), jnp.float32)] * 3,
)


def _step_body(pa_ref, pb_ref, d_ref, w_ref, yin_ref, y_ref, u_ref):
    d = d_ref[...]
    h = d * (pa_ref[0] + pb_ref[0])
    y_ref[...] = yin_ref[...] + jnp.dot(
        h, w_ref[...], preferred_element_type=jnp.float32)
    u_ref[...] = d * h


_tc_step = pl.pallas_call(
    _step_body,
    grid=(GRID,),
    in_specs=[_part(FD, 0), _part(FD, 1), _rows(FD), _full((FD, FD)), _rows(FD)],
    out_specs=[_rows(FD), _rows(FD)],
    out_shape=[jax.ShapeDtypeStruct((N, FD), jnp.float32)] * 2,
)


def _bridge_body(pa_ref, pb_ref, d_ref, w_ref, yin_ref, b_ref, wn_ref,
                 y_ref, u_ref):
    d = d_ref[...]
    h = d * (pa_ref[0] + pb_ref[0])
    a = jnp.maximum(
        yin_ref[...]
        + jnp.dot(h, w_ref[...], preferred_element_type=jnp.float32)
        + b_ref[...], 0.0)
    y_ref[...] = jnp.dot(a, wn_ref[...], preferred_element_type=jnp.float32)
    u_ref[...] = d * a


_tc_bridge = pl.pallas_call(
    _bridge_body,
    grid=(GRID,),
    in_specs=[_part(FD, 0), _part(FD, 1), _rows(FD), _full((FD, FD)), _rows(FD),
              _full((1, FD)), _full((FD, FD))],
    out_specs=[_rows(FD), _rows(FD)],
    out_shape=[jax.ShapeDtypeStruct((N, FD), jnp.float32)] * 2,
)


def _final_body(pa_ref, pb_ref, d_ref, w_ref, yin_ref, b_ref, wc_ref, bc_ref,
                o_ref):
    d = d_ref[...]
    h = d * (pa_ref[0] + pb_ref[0])
    a = jnp.maximum(
        yin_ref[...]
        + jnp.dot(h, w_ref[...], preferred_element_type=jnp.float32)
        + b_ref[...], 0.0)
    o_ref[...] = jnp.dot(
        a, wc_ref[...], preferred_element_type=jnp.float32) + bc_ref[...]


_tc_final = pl.pallas_call(
    _final_body,
    grid=(GRID,),
    in_specs=[_part(FD, 0), _part(FD, 1), _rows(FD), _full((FD, FD)), _rows(FD),
              _full((1, FD)), _full((FD, NCLS)), _full((1, NCLS))],
    out_specs=_rows(NCLS),
    out_shape=jax.ShapeDtypeStruct((N, NCLS), jnp.float32),
)


def kernel(x, edge_index, W1, b1, W2, b2, Wc, bc):
    ei = edge_index.astype(jnp.int32)
    src = ei[0]
    dst = ei[1]

    dst_deg = dst.reshape(NW, NCHD, BD)

    degp = _sc_degree(dst_deg)
    y, u, dinvb = _tc_prep(x, degp, degp, W1[0])

    for k in (1, 2):
        p = _sc_propagate(u, src, dst)
        y, u = _tc_step(p, p, dinvb, W1[k], y)
    p = _sc_propagate(u, src, dst)
    y, u = _tc_bridge(p, p, dinvb, W1[3], y, b1.reshape(1, FD), W2[0])

    for k in (1, 2):
        p = _sc_propagate(u, src, dst)
        y, u = _tc_step(p, p, dinvb, W2[k], y)
    p = _sc_propagate(u, src, dst)
    return _tc_final(p, p, dinvb, W2[3], y, b2.reshape(1, FD),
                     Wc, bc.reshape(1, NCLS))
